# Initial kernel scaffold; baseline (speedup 1.0000x reference)
import jax
import jax.numpy as jnp
from jax.experimental import pallas as pl


def kernel(x, edge_index, edge_attr, batch, dataset_graph_idx, We, be, W, b, Wp, bp):
    raise NotImplementedError("write your pallas kernel here")


# speedup vs baseline: 1.4436x; 1.4436x over previous
import functools

import jax
import jax.numpy as jnp
from jax import lax
from jax.experimental import pallas as pl
from jax.experimental.pallas import tpu as pltpu
from jax.experimental.pallas import tpu_sc as plsc

N_NODES = 10000
N_EDGES = 160000
EMB = 300
PROJ = 100
D_EDGE = 16
NUM_GRAPHS = 512
NUM_LAYERS = 5
INV_T = 25.0

CW = 128
NCHUNK = 3
EMBP = CW * NCHUNK
N_TILES = 16
TPE = N_EDGES // N_TILES
SC_K = 80
NW = TPE // SC_K
AGGN = 10240
RPT = AGGN // N_TILES
ZR = 128

TE = 2000
TN = 1000


def _edge_body(ea_ref, we_ref, be_ref, e0_ref, e1_ref, e2_ref):
    e = jnp.dot(ea_ref[...], we_ref[...], preferred_element_type=jnp.float32)
    e = jnp.maximum(e + be_ref[...], 0.0)
    e0_ref[...] = e[:, :CW]
    e1_ref[...] = e[:, CW:2 * CW]
    e2_ref[...] = e[:, 2 * CW:]


def _tc_edge(ea, we_l, be_l):
    return pl.pallas_call(
        _edge_body,
        grid=(N_EDGES // TE,),
        in_specs=[
            pl.BlockSpec((TE, D_EDGE), lambda i: (i, 0)),
            pl.BlockSpec((D_EDGE, EMBP), lambda i: (0, 0)),
            pl.BlockSpec((1, EMBP), lambda i: (0, 0)),
        ],
        out_specs=[pl.BlockSpec((TE, CW), lambda i: (i, 0))] * NCHUNK,
        out_shape=[jax.ShapeDtypeStruct((N_EDGES, CW), jnp.float32)] * NCHUNK,
    )(ea, we_l, be_l)


def _node_body(h0_ref, h1_ref, h2_ref, a0_ref, a1_ref, a2_ref, w_ref, b_ref,
               o0_ref, o1_ref, o2_ref, *, relu):
    hcat = jnp.concatenate(
        [h0_ref[...] + a0_ref[...], h1_ref[...] + a1_ref[...],
         h2_ref[...] + a2_ref[...]], axis=1)
    out = jnp.dot(hcat, w_ref[...], preferred_element_type=jnp.float32)
    out = out + b_ref[...]
    if relu:
        out = jnp.maximum(out, 0.0)
    o0_ref[...] = out[:, :CW]
    o1_ref[...] = out[:, CW:2 * CW]
    o2_ref[...] = out[:, 2 * CW:]


def _tc_node(hs, aggs, w_l, b_l, relu):
    return pl.pallas_call(
        functools.partial(_node_body, relu=relu),
        grid=(N_NODES // TN,),
        in_specs=(
            [pl.BlockSpec((TN, CW), lambda i: (i, 0))] * (2 * NCHUNK)
            + [pl.BlockSpec((EMBP, EMBP), lambda i: (0, 0)),
               pl.BlockSpec((1, EMBP), lambda i: (0, 0))]
        ),
        out_specs=[pl.BlockSpec((TN, CW), lambda i: (i, 0))] * NCHUNK,
        out_shape=[jax.ShapeDtypeStruct((N_NODES, CW), jnp.float32)] * NCHUNK,
    )(*hs, *aggs, w_l, b_l)


def _final_body(h0_ref, h1_ref, h2_ref, wp_ref, bp_ref, batch_ref, out_ref,
                pooled_acc, cnt_acc):
    i = pl.program_id(0)

    @pl.when(i == 0)
    def _():
        pooled_acc[...] = jnp.zeros_like(pooled_acc)
        cnt_acc[...] = jnp.zeros_like(cnt_acc)

    hcat = jnp.concatenate([h0_ref[...], h1_ref[...], h2_ref[...]], axis=1)
    o = jnp.dot(hcat, wp_ref[...], preferred_element_type=jnp.float32)
    o = o + bp_ref[...]
    onehot = (batch_ref[0] ==
              lax.broadcasted_iota(jnp.int32, (NUM_GRAPHS, TN), 0)
              ).astype(jnp.float32)
    pooled_acc[...] += lax.dot_general(
        onehot, o, (((1,), (0,)), ((), ())),
        preferred_element_type=jnp.float32)
    cnt_acc[...] += jnp.sum(onehot, axis=1, keepdims=True)

    @pl.when(i == pl.num_programs(0) - 1)
    def _():
        pooled = pooled_acc[...] / jnp.maximum(cnt_acc[...], 1.0)
        norm = jnp.sqrt(jnp.sum(pooled * pooled, axis=1, keepdims=True))
        f = pooled / jnp.maximum(norm, 1e-12)
        n = NUM_GRAPHS // 2
        out_ref[...] = lax.dot_general(
            f[:n], f[n:], (((1,), (1,)), ((), ())),
            preferred_element_type=jnp.float32) * INV_T


def _tc_final(hs, wp, bp, batch3d):
    n = NUM_GRAPHS // 2
    return pl.pallas_call(
        _final_body,
        grid=(N_NODES // TN,),
        in_specs=[
            pl.BlockSpec((TN, CW), lambda i: (i, 0)),
            pl.BlockSpec((TN, CW), lambda i: (i, 0)),
            pl.BlockSpec((TN, CW), lambda i: (i, 0)),
            pl.BlockSpec((EMBP, PROJ), lambda i: (0, 0)),
            pl.BlockSpec((1, PROJ), lambda i: (0, 0)),
            pl.BlockSpec((1, 1, TN), lambda i: (i, 0, 0)),
        ],
        out_specs=pl.BlockSpec((n, n), lambda i: (0, 0)),
        out_shape=jax.ShapeDtypeStruct((n, n), jnp.float32),
        scratch_shapes=[
            pltpu.VMEM((NUM_GRAPHS, PROJ), jnp.float32),
            pltpu.VMEM((NUM_GRAPHS, 1), jnp.float32),
        ],
    )(*hs, wp, bp, batch3d)


def _sc_body(h0, h1, h2, e0, e1, e2, src_hbm, dst_hbm, agg0, agg1, agg2,
             idx_v, dsti_v, e_v, h_v, zbuf, agg_sh, sem):
    c = lax.axis_index("c")
    s = lax.axis_index("s")
    zero = jnp.zeros((16,), jnp.float32)

    @pl.loop(0, ZR)
    def _(r):
        for j in range(CW // 16):
            zbuf[r, pl.ds(j * 16, 16)] = zero

    def one_pass(h_ref, e_ref, out_ref):
        for j in range(RPT // ZR):
            pltpu.sync_copy(zbuf, agg_sh.at[pl.ds(s * RPT + j * ZR, ZR)])
        plsc.subcore_barrier()

        base_t = s * TPE

        @pl.loop(0, NW)
        def _(w):
            base = base_t + w * SC_K
            pltpu.sync_copy(src_hbm.at[pl.ds(base, SC_K)], idx_v)
            pltpu.sync_copy(dst_hbm.at[pl.ds(base, SC_K)], dsti_v)
            pltpu.async_copy(h_ref.at[idx_v], h_v, sem).wait()
            pltpu.sync_copy(e_ref.at[pl.ds(base, SC_K)], e_v)

            @pl.loop(0, SC_K)
            def _(r):
                for j in range(CW // 16):
                    sl = pl.ds(j * 16, 16)
                    e_v[r, sl] = jnp.maximum(e_v[r, sl] + h_v[r, sl], 0.0)

            pltpu.sync_copy(e_v, agg_sh.at[dsti_v], add=True)

        plsc.subcore_barrier()
        for j in range(RPT // ZR):
            sl = pl.ds(s * RPT + j * ZR, ZR)
            pltpu.sync_copy(agg_sh.at[sl], out_ref.at[sl])
        plsc.subcore_barrier()

    @pl.when(c == 0)
    def _():
        one_pass(h0, e0, agg0)

    @pl.when(c == 1)
    def _():
        one_pass(h1, e1, agg1)
        one_pass(h2, e2, agg2)


@functools.cache
def _sc_agg_call():
    return pl.kernel(
        _sc_body,
        out_type=tuple(
            jax.ShapeDtypeStruct((AGGN, CW), jnp.float32)
            for _ in range(NCHUNK)),
        mesh=plsc.VectorSubcoreMesh(core_axis_name="c", subcore_axis_name="s",
                                    num_cores=2, num_subcores=16),
        scratch_types=[
            pltpu.VMEM((SC_K,), jnp.int32),
            pltpu.VMEM((SC_K,), jnp.int32),
            pltpu.VMEM((SC_K, CW), jnp.float32),
            pltpu.VMEM((SC_K, CW), jnp.float32),
            pltpu.VMEM((ZR, CW), jnp.float32),
            pltpu.VMEM_SHARED((AGGN, CW), jnp.float32),
            pltpu.SemaphoreType.DMA,
        ],
    )


def _sc_agg(hs, es, src, dst):
    return _sc_agg_call()(*hs, *es, src, dst)


def kernel(x, edge_index, edge_attr, batch, dataset_graph_idx,
           We, be, W, b, Wp, bp):
    src = edge_index[0]
    dst = edge_index[1]
    padw = EMBP - EMB

    x_p = jnp.pad(x, ((0, 0), (0, padw)))
    We_p = jnp.pad(We, ((0, 0), (0, 0), (0, padw)))
    be_p = jnp.pad(be, ((0, 0), (0, padw)))[:, None, :]
    W_p = jnp.pad(W, ((0, 0), (0, padw), (0, padw)))
    b_p = jnp.pad(b, ((0, 0), (0, padw)))[:, None, :]
    Wp_p = jnp.pad(Wp, ((0, padw), (0, 0)))
    bp_p = bp[None, :]

    hs = (x_p[:, :CW], x_p[:, CW:2 * CW], x_p[:, 2 * CW:])
    for l in range(NUM_LAYERS):
        es = _tc_edge(edge_attr, We_p[l], be_p[l])
        aggs = _sc_agg(hs, es, src, dst)
        hs = _tc_node(hs, aggs, W_p[l], b_p[l], relu=(l < NUM_LAYERS - 1))

    batch3d = batch.reshape(N_NODES // TN, 1, TN)
    logits = _tc_final(hs, Wp_p, bp_p, batch3d)
    labels = jnp.arange(NUM_GRAPHS // 2, dtype=jnp.int32)
    return logits, labels
